# Initial kernel scaffold; baseline (speedup 1.0000x reference)
import jax
import jax.numpy as jnp
from jax.experimental import pallas as pl


def kernel(nodes_features, edge_index, W_self1, W_neigh1, b1, W_self2, W_neigh2, b2):
    raise NotImplementedError("write your pallas kernel here")


# speedup vs baseline: 2.6156x; 2.6156x over previous
import functools

import jax
import jax.numpy as jnp
from jax import lax
from jax.experimental import pallas as pl
from jax.experimental.pallas import tpu as pltpu
from jax.experimental.pallas import tpu_sc as plsc

N = 10000
E = 160000
D = 256
HALF = 128

NC = 2
NS = 16
CH = 128

EPS = E // NS
EP = ((EPS + 2 * CH - 1) // (2 * CH)) * (2 * CH)
NCHUNK = EP // CH
NP2 = 10112
NRI = NP2 // NS
DEGR = 80

BR = 1000
NBLK = N // BR


def _make_sc_agg():
    mesh = plsc.VectorSubcoreMesh(core_axis_name="c", subcore_axis_name="s")
    agg_t = jax.ShapeDtypeStruct((2 * NP2, HALF), jnp.float32)
    scratch = [
        pltpu.VMEM((CH,), jnp.int32),
        pltpu.VMEM((CH,), jnp.int32),
        pltpu.VMEM((CH, HALF), jnp.float32),
        pltpu.VMEM_SHARED((NP2, HALF), jnp.float32),
        pltpu.SemaphoreType.DMA,
    ]

    @functools.partial(pl.kernel, mesh=mesh, out_type=agg_t,
                       scratch_types=scratch)
    def k(tables, srcp2, dstp, zrows,
          agg_out, src_v, dst_v, rows_v, acc_sh, sem):
        c = lax.axis_index("c")
        s = lax.axis_index("s")
        r0i = s * NRI
        pltpu.sync_copy(zrows.at[pl.ds(r0i, NRI)],
                        acc_sh.at[pl.ds(r0i, NRI)])
        plsc.subcore_barrier()
        ebase = c * (NS * EP) + s * EP
        dbase = s * EP

        def chunk(i, carry):
            pltpu.sync_copy(srcp2.at[pl.ds(ebase + i * CH, CH)], src_v)
            pltpu.sync_copy(dstp.at[pl.ds(dbase + i * CH, CH)], dst_v)
            pltpu.async_copy(tables.at[src_v], rows_v, sem).wait()
            pltpu.sync_copy(rows_v, acc_sh.at[dst_v], add=True)
            return carry

        lax.fori_loop(0, NCHUNK, chunk, 0)
        plsc.subcore_barrier()
        r0 = s * NRI
        pltpu.sync_copy(acc_sh.at[pl.ds(r0, NRI)],
                        agg_out.at[pl.ds(c * NP2 + r0, NRI)])

    return k


def _make_sc_deg():
    mesh = plsc.VectorSubcoreMesh(core_axis_name="c", subcore_axis_name="s")
    deg_t = jax.ShapeDtypeStruct((2 * NP2, HALF), jnp.float32)
    scratch = [
        pltpu.VMEM((CH,), jnp.int32),
        pltpu.VMEM((CH, HALF), jnp.float32),
        pltpu.VMEM_SHARED((NP2, HALF), jnp.float32),
        pltpu.SemaphoreType.DMA,
    ]
    ndeg = NCHUNK // 2

    @functools.partial(pl.kernel, mesh=mesh, out_type=deg_t,
                       scratch_types=scratch)
    def k(dstp, zrows, ones_h, deg_out, dst_v, ones_v, deg_sh, sem):
        c = lax.axis_index("c")
        s = lax.axis_index("s")
        r0i = s * NRI
        pltpu.sync_copy(zrows.at[pl.ds(r0i, NRI)],
                        deg_sh.at[pl.ds(r0i, NRI)])
        pltpu.sync_copy(ones_h, ones_v)
        plsc.subcore_barrier()
        dbase = s * EP + c * (ndeg * CH)

        def chunk(i, carry):
            pltpu.sync_copy(dstp.at[pl.ds(dbase + i * CH, CH)], dst_v)
            pltpu.sync_copy(ones_v, deg_sh.at[dst_v], add=True)
            return carry

        lax.fori_loop(0, ndeg, chunk, 0)
        plsc.subcore_barrier()
        r0 = s * NRI
        pltpu.sync_copy(deg_sh.at[pl.ds(r0, NRI)],
                        deg_out.at[pl.ds(c * NP2 + r0, NRI)])

    return k


_sc_agg = _make_sc_agg()
_sc_deg = _make_sc_deg()


def _tc_layer_body(relu, x_ref, a0_ref, a1_ref, d0_ref, d1_ref, ws_ref,
                   wn_ref, b_ref, out_ref):
    r = 1.0 / jnp.maximum(d0_ref[...] + d1_ref[...], 1.0)
    h = jnp.dot(x_ref[...], ws_ref[...], preferred_element_type=jnp.float32)
    h = h + jnp.dot(a0_ref[...] * r, wn_ref[0:HALF, :],
                    preferred_element_type=jnp.float32)
    h = h + jnp.dot(a1_ref[...] * r, wn_ref[HALF:D, :],
                    preferred_element_type=jnp.float32)
    h = h + b_ref[...]
    if relu:
        h = jnp.maximum(h, 0.0)
    out_ref[...] = h


def _tc_layer(x, agg, degs, ws, wn, b2d, relu):
    a0 = lax.slice(agg, (0, 0), (N, HALF))
    a1 = lax.slice(agg, (NP2, 0), (NP2 + N, HALF))
    d0 = lax.slice(degs, (0, 0), (N, 1))
    d1 = lax.slice(degs, (NP2, 0), (NP2 + N, 1))
    grid = (NBLK,)
    in_specs = [
        pl.BlockSpec((BR, D), lambda i: (i, 0)),
        pl.BlockSpec((BR, HALF), lambda i: (i, 0)),
        pl.BlockSpec((BR, HALF), lambda i: (i, 0)),
        pl.BlockSpec((BR, 1), lambda i: (i, 0)),
        pl.BlockSpec((BR, 1), lambda i: (i, 0)),
        pl.BlockSpec((D, D), lambda i: (0, 0)),
        pl.BlockSpec((D, D), lambda i: (0, 0)),
        pl.BlockSpec((1, D), lambda i: (0, 0)),
    ]
    return pl.pallas_call(
        functools.partial(_tc_layer_body, relu),
        grid=grid,
        in_specs=in_specs,
        out_specs=pl.BlockSpec((BR, D), lambda i: (i, 0)),
        out_shape=jax.ShapeDtypeStruct((N, D), jnp.float32),
    )(x, a0, a1, d0, d1, ws, wn, b2d)


def kernel(nodes_features, edge_index, W_self1, W_neigh1, b1,
           W_self2, W_neigh2, b2):
    x = nodes_features
    src = edge_index[0].astype(jnp.int32)
    dst = edge_index[1].astype(jnp.int32)

    pad = EP - EPS
    src_p = jnp.pad(src.reshape(NS, EPS), ((0, 0), (0, pad)))
    pad_dst = jnp.broadcast_to(
        N + (jnp.arange(pad, dtype=jnp.int32) % 16), (NS, pad))
    dst_p = jnp.concatenate([dst.reshape(NS, EPS), pad_dst], axis=1)
    dstp = dst_p.reshape(-1)
    s2 = (2 * src_p).reshape(-1)
    srcp2 = jnp.concatenate([s2, s2 + 1])

    zrows = jnp.zeros((NP2, HALF), jnp.float32)
    ones_h = jnp.ones((CH, HALF), jnp.float32)
    b1_2d = b1.reshape(1, D)
    b2_2d = b2.reshape(1, D)

    tables1 = x.reshape(2 * N, HALF)
    degs = _sc_deg(dstp, zrows, ones_h)
    agg1 = _sc_agg(tables1, srcp2, dstp, zrows)
    h = _tc_layer(x, agg1, degs, W_self1, W_neigh1, b1_2d, relu=True)

    tables2 = h.reshape(2 * N, HALF)
    agg2 = _sc_agg(tables2, srcp2, dstp, zrows)
    out = _tc_layer(h, agg2, degs, W_self2, W_neigh2, b2_2d, relu=False)
    return out


# speedup vs baseline: 3.7712x; 1.4418x over previous
import functools

import jax
import jax.numpy as jnp
from jax import lax
from jax.experimental import pallas as pl
from jax.experimental.pallas import tpu as pltpu
from jax.experimental.pallas import tpu_sc as plsc

N = 10000
E = 160000
D = 256
HALF = 128

NC = 2
NS = 16
CH = 128

EPS = E // NS
EP = ((EPS + 2 * CH - 1) // (2 * CH)) * (2 * CH)
NCHUNK = EP // CH
NP2 = 10112
NRI = NP2 // NS
DEGR = 80

BR = 1000
NBLK = N // BR


def _make_sc_agg():
    mesh = plsc.VectorSubcoreMesh(core_axis_name="c", subcore_axis_name="s")
    agg_t = jax.ShapeDtypeStruct((2 * NP2, HALF), jnp.float32)
    PH = NCHUNK // 2
    scratch = [
        pltpu.VMEM((NCHUNK, CH), jnp.int32),
        pltpu.VMEM((PH, CH), jnp.int32),
        pltpu.VMEM((2, CH, HALF), jnp.float32),
        pltpu.VMEM_SHARED((NP2, HALF), jnp.float32),
        pltpu.SemaphoreType.DMA,
    ]

    @functools.partial(pl.kernel, mesh=mesh, out_type=agg_t,
                       scratch_types=scratch)
    def k(tables, srcp3, dstp3, zrows,
          agg_out, sidx, didx, rows, acc_sh, sem):
        c = lax.axis_index("c")
        s = lax.axis_index("s")
        r0i = s * NRI
        pltpu.sync_copy(zrows.at[pl.ds(r0i, NRI)],
                        acc_sh.at[pl.ds(r0i, NRI)])
        pltpu.sync_copy(srcp3.at[c * NS + s], sidx)
        plsc.subcore_barrier()

        pltpu.async_copy(tables.at[sidx.at[0]], rows.at[0], sem)
        pltpu.async_copy(tables.at[sidx.at[1]], rows.at[1], sem)

        for phase in range(2):
            pltpu.sync_copy(dstp3.at[s, pl.ds(phase * PH, PH)], didx)

            def chunk2(t, carry):
                for b in range(2):
                    li = 2 * t + b
                    g = phase * PH + li

                    pltpu.make_async_copy(
                        tables.at[sidx.at[0]], rows.at[b], sem).wait()
                    pltpu.sync_copy(rows.at[b], acc_sh.at[didx.at[li]],
                                    add=True)

                    @pl.when(g + 2 < NCHUNK)
                    def _():
                        pltpu.async_copy(
                            tables.at[sidx.at[g + 2]], rows.at[b], sem)
                return carry

            lax.fori_loop(0, PH // 2, chunk2, 0)
        plsc.subcore_barrier()
        r0 = s * NRI
        pltpu.sync_copy(acc_sh.at[pl.ds(r0, NRI)],
                        agg_out.at[pl.ds(c * NP2 + r0, NRI)])

    return k


def _make_sc_deg():
    mesh = plsc.VectorSubcoreMesh(core_axis_name="c", subcore_axis_name="s")
    deg_t = jax.ShapeDtypeStruct((2 * NP2, HALF), jnp.float32)
    scratch = [
        pltpu.VMEM((NCHUNK, CH), jnp.int32),
        pltpu.VMEM((CH, HALF), jnp.float32),
        pltpu.VMEM_SHARED((NP2, HALF), jnp.float32),
        pltpu.SemaphoreType.DMA,
    ]
    ndeg = NCHUNK // 2

    @functools.partial(pl.kernel, mesh=mesh, out_type=deg_t,
                       scratch_types=scratch)
    def k(dstp3, zrows, ones_h, deg_out, didx, ones_v, deg_sh, sem):
        c = lax.axis_index("c")
        s = lax.axis_index("s")
        r0i = s * NRI
        pltpu.sync_copy(zrows.at[pl.ds(r0i, NRI)],
                        deg_sh.at[pl.ds(r0i, NRI)])
        pltpu.sync_copy(dstp3.at[s], didx)
        pltpu.sync_copy(ones_h, ones_v)
        plsc.subcore_barrier()

        def chunk(i, carry):
            pltpu.sync_copy(ones_v, deg_sh.at[didx.at[c * ndeg + i]],
                            add=True)
            return carry

        lax.fori_loop(0, ndeg, chunk, 0)
        plsc.subcore_barrier()
        r0 = s * NRI
        pltpu.sync_copy(deg_sh.at[pl.ds(r0, NRI)],
                        deg_out.at[pl.ds(c * NP2 + r0, NRI)])

    return k


_sc_agg = _make_sc_agg()
_sc_deg = _make_sc_deg()


def _tc_layer_body(relu, x_ref, a0_ref, a1_ref, d0_ref, d1_ref, ws_ref,
                   wn_ref, b_ref, out_ref):
    r = 1.0 / jnp.maximum(d0_ref[...] + d1_ref[...], 1.0)
    h = jnp.dot(x_ref[...], ws_ref[...], preferred_element_type=jnp.float32)
    h = h + jnp.dot(a0_ref[...] * r, wn_ref[0:HALF, :],
                    preferred_element_type=jnp.float32)
    h = h + jnp.dot(a1_ref[...] * r, wn_ref[HALF:D, :],
                    preferred_element_type=jnp.float32)
    h = h + b_ref[...]
    if relu:
        h = jnp.maximum(h, 0.0)
    out_ref[...] = h


def _tc_layer(x, agg, degs, ws, wn, b2d, relu):
    a0 = lax.slice(agg, (0, 0), (N, HALF))
    a1 = lax.slice(agg, (NP2, 0), (NP2 + N, HALF))
    d0 = lax.slice(degs, (0, 0), (N, 1))
    d1 = lax.slice(degs, (NP2, 0), (NP2 + N, 1))
    grid = (NBLK,)
    in_specs = [
        pl.BlockSpec((BR, D), lambda i: (i, 0)),
        pl.BlockSpec((BR, HALF), lambda i: (i, 0)),
        pl.BlockSpec((BR, HALF), lambda i: (i, 0)),
        pl.BlockSpec((BR, 1), lambda i: (i, 0)),
        pl.BlockSpec((BR, 1), lambda i: (i, 0)),
        pl.BlockSpec((D, D), lambda i: (0, 0)),
        pl.BlockSpec((D, D), lambda i: (0, 0)),
        pl.BlockSpec((1, D), lambda i: (0, 0)),
    ]
    return pl.pallas_call(
        functools.partial(_tc_layer_body, relu),
        grid=grid,
        in_specs=in_specs,
        out_specs=pl.BlockSpec((BR, D), lambda i: (i, 0)),
        out_shape=jax.ShapeDtypeStruct((N, D), jnp.float32),
    )(x, a0, a1, d0, d1, ws, wn, b2d)


def kernel(nodes_features, edge_index, W_self1, W_neigh1, b1,
           W_self2, W_neigh2, b2):
    x = nodes_features
    src = edge_index[0].astype(jnp.int32)
    dst = edge_index[1].astype(jnp.int32)

    pad = EP - EPS
    src_p = jnp.pad(src.reshape(NS, EPS), ((0, 0), (0, pad)))
    pad_dst = jnp.broadcast_to(
        N + (jnp.arange(pad, dtype=jnp.int32) % 16), (NS, pad))
    dst_p = jnp.concatenate([dst.reshape(NS, EPS), pad_dst], axis=1)
    dstp = dst_p.reshape(NS, NCHUNK, CH)
    s2 = 2 * src_p
    srcp2 = jnp.concatenate([s2, s2 + 1]).reshape(2 * NS, NCHUNK, CH)

    zrows = jnp.zeros((NP2, HALF), jnp.float32)
    ones_h = jnp.ones((CH, HALF), jnp.float32)
    b1_2d = b1.reshape(1, D)
    b2_2d = b2.reshape(1, D)

    tables1 = x.reshape(2 * N, HALF)
    degs = _sc_deg(dstp, zrows, ones_h)
    agg1 = _sc_agg(tables1, srcp2, dstp, zrows)
    h = _tc_layer(x, agg1, degs, W_self1, W_neigh1, b1_2d, relu=True)

    tables2 = h.reshape(2 * N, HALF)
    agg2 = _sc_agg(tables2, srcp2, dstp, zrows)
    out = _tc_layer(h, agg2, degs, W_self2, W_neigh2, b2_2d, relu=False)
    return out
